# baseline (speedup 1.0000x reference)
import jax
import jax.numpy as jnp
from jax.experimental import pallas as pl
from jax.experimental.pallas import tpu as pltpu

N = 10000
E = 320000
H = 128
HEADS = 2
G = 64


def _gcn(x, W, b, row, col):
    h = x @ W
    deg = jnp.zeros((N,), jnp.float32).at[col].add(1.0)
    dis = jnp.where(deg > 0, 1.0 / jnp.sqrt(deg), 0.0)
    norm = dis[row] * dis[col]
    out = jnp.zeros((N, W.shape[1]), jnp.float32).at[col].add(norm[:, None] * h[row])
    return out + b


def _gat(x, W, a_src, a_dst, b, row, col):
    C = W.shape[1] // HEADS
    h = (x @ W).reshape(N, HEADS, C)
    als = jnp.sum(h * a_src[None], axis=-1)
    ald = jnp.sum(h * a_dst[None], axis=-1)
    e = jax.nn.leaky_relu(als[row] + ald[col], 0.2)
    emax = jax.ops.segment_max(e, col, num_segments=N)
    ex = jnp.exp(e - emax[col])
    den = jax.ops.segment_sum(ex, col, num_segments=N)
    alpha = ex / (den[col] + 1e-16)
    out = jax.ops.segment_sum(alpha[:, :, None] * h[row], col, num_segments=N)
    return out.mean(axis=1) + b


def _bn(x, g, b):
    return x * (g / jnp.sqrt(1.0 + 1e-5)) + b


def _mlp_kernel(xc_ref, cW1_ref, cb1_ref, cbn_g_ref, cbn_b_ref, cW2_ref,
                cb2_ref, cln_g_ref, cln_b_ref, cW3_ref, cb3_ref, out_ref):
    xc = xc_ref[...]
    z = jnp.dot(xc, cW1_ref[...], preferred_element_type=jnp.float32) + cb1_ref[...]
    z = z * (cbn_g_ref[...] / jnp.sqrt(1.0 + 1e-5)) + cbn_b_ref[...]
    z = jnp.maximum(z, 0.0)
    z = jnp.dot(z, cW2_ref[...], preferred_element_type=jnp.float32) + cb2_ref[...]
    m = jnp.mean(z, axis=-1, keepdims=True)
    v = jnp.mean(jnp.square(z - m), axis=-1, keepdims=True)
    z = (z - m) / jnp.sqrt(v + 1e-5) * cln_g_ref[...] + cln_b_ref[...]
    z = jnp.maximum(z, 0.0)
    out_ref[...] = jnp.dot(z, cW3_ref[...], preferred_element_type=jnp.float32) + cb3_ref[...]


def kernel(x, edge_index, batch, W0, b0, Wg1, asrc1, adst1, bg1, W2c, b2c, Wg3, asrc3, adst3, bg3, W4, b4, bn_gamma, bn_beta, cW1, cb1, cbn_g, cbn_b, cW2, cb2, cln_g, cln_b, cW3, cb3):
    loop = jnp.arange(N, dtype=edge_index.dtype)
    row = jnp.concatenate([edge_index[0], loop])
    col = jnp.concatenate([edge_index[1], loop])

    h = jax.nn.relu(_bn(_gcn(x, W0, b0, row, col), bn_gamma[0], bn_beta[0]))
    prev = h
    h = jax.nn.relu(_bn(_gat(h, Wg1, asrc1, adst1, bg1, row, col), bn_gamma[1], bn_beta[1])) + prev
    prev = h
    h = jax.nn.relu(_bn(_gcn(h, W2c, b2c, row, col), bn_gamma[2], bn_beta[2])) + prev
    prev = h
    h = jax.nn.relu(_bn(_gat(h, Wg3, asrc3, adst3, bg3, row, col), bn_gamma[3], bn_beta[3])) + prev
    prev = h
    h = jax.nn.relu(_bn(_gcn(h, W4, b4, row, col), bn_gamma[4], bn_beta[4])) + prev

    counts = jax.ops.segment_sum(jnp.ones((N,), jnp.float32), batch, num_segments=G)
    xmean = jax.ops.segment_sum(h, batch, num_segments=G) / jnp.maximum(counts, 1.0)[:, None]
    xmax = jax.ops.segment_max(h, batch, num_segments=G)
    xc = jnp.concatenate([xmean, xmax], axis=1)

    return pl.pallas_call(
        _mlp_kernel,
        out_shape=jax.ShapeDtypeStruct((G, 2), jnp.float32),
    )(xc, cW1, cb1, cbn_g, cbn_b, cW2, cb2, cln_g, cln_b, cW3, cb3)


# speedup vs baseline: 2.5404x; 2.5404x over previous
import functools

import jax
import jax.numpy as jnp
import numpy as _np
from jax import lax
from jax.experimental import pallas as pl
from jax.experimental.pallas import tpu as pltpu
from jax.experimental.pallas import tpu_sc as plsc

N = 10000
E = 320000
H = 128
HEADS = 2
G = 64

NPAD = 10240
E2 = E + N
NC, NS = 2, 16
NW = NC * NS
K = 128
NCH = -(-E2 // (NW * K))
EPT = NCH * K
EPAD = NW * EPT
STRIPE = NPAD // NS
RB = 512
GRID = NPAD // RB

_i32 = jnp.int32

_mesh = plsc.VectorSubcoreMesh(core_axis_name="c", subcore_axis_name="s")


def _zero_acc(buf, acc, sid):
    zero16 = jnp.zeros((16,), jnp.float32)

    def zbody(r, carry):
        for c in range(H // 16):
            buf[r, pl.ds(c * 16, 16)] = zero16
        return carry

    lax.fori_loop(0, K, zbody, 0)
    for kk in range(STRIPE // K):
        pltpu.sync_copy(buf, acc.at[pl.ds(sid * STRIPE + kk * K, K)])


@functools.partial(
    pl.kernel,
    out_type=jax.ShapeDtypeStruct((NC, NPAD, H), jnp.float32),
    mesh=_mesh,
    scratch_types=[
        pltpu.VMEM((K,), _i32),
        pltpu.VMEM((1, K), _i32),
        pltpu.VMEM((K, H), jnp.float32),
        pltpu.VMEM_SHARED((NPAD, H), jnp.float32),
        pltpu.SemaphoreType.DMA,
    ],
)
def _sc_agg(src_hbm, row_hbm, col_hbm, out_hbm, row_c, col_c, buf, acc, sem):
    cid = lax.axis_index("c")
    sid = lax.axis_index("s")
    wid = sid * NC + cid
    _zero_acc(buf, acc, sid)
    plsc.subcore_barrier()

    def body(j, carry):
        pltpu.sync_copy(row_hbm.at[wid, j], row_c)
        pltpu.sync_copy(col_hbm.at[wid, j], col_c)
        pltpu.async_copy(src_hbm.at[row_c], buf, sem).wait()
        pltpu.sync_copy(buf, acc.at[col_c.at[0]], add=True)
        return carry

    lax.fori_loop(0, NCH, body, 0)
    plsc.subcore_barrier()
    pltpu.sync_copy(acc.at[pl.ds(sid * STRIPE, STRIPE)],
                    out_hbm.at[cid, pl.ds(sid * STRIPE, STRIPE)])


@functools.partial(
    pl.kernel,
    out_type=jax.ShapeDtypeStruct((NC, NPAD, H), jnp.float32),
    mesh=_mesh,
    scratch_types=[
        pltpu.VMEM((K,), _i32),
        pltpu.VMEM((1, K), _i32),
        pltpu.VMEM((K, 16), jnp.float32),
        pltpu.VMEM((K, H), jnp.float32),
        pltpu.VMEM_SHARED((NPAD, H), jnp.float32),
        pltpu.SemaphoreType.DMA,
    ],
)
def _sc_agg_w(src_hbm, row_hbm, col_hbm, w_hbm, out_hbm,
              row_c, col_c, wbuf, buf, acc, sem):
    cid = lax.axis_index("c")
    sid = lax.axis_index("s")
    wid = sid * NC + cid
    _zero_acc(buf, acc, sid)
    plsc.subcore_barrier()

    def body(j, carry):
        pltpu.sync_copy(row_hbm.at[wid, j], row_c)
        pltpu.sync_copy(col_hbm.at[wid, j], col_c)
        pltpu.sync_copy(w_hbm.at[wid, j], wbuf)
        pltpu.async_copy(src_hbm.at[row_c], buf, sem).wait()

        def rowbody(r, c2):
            wspl = wbuf[r]
            for c in range(H // 16):
                sl = pl.ds(c * 16, 16)
                buf[r, sl] = buf[r, sl] * wspl
            return c2

        lax.fori_loop(0, K, rowbody, 0)
        pltpu.sync_copy(buf, acc.at[col_c.at[0]], add=True)
        return carry

    lax.fori_loop(0, NCH, body, 0)
    plsc.subcore_barrier()
    pltpu.sync_copy(acc.at[pl.ds(sid * STRIPE, STRIPE)],
                    out_hbm.at[cid, pl.ds(sid * STRIPE, STRIPE)])


def _mm_scale_body(x_ref, w_ref, s_ref, o_ref):
    o_ref[...] = jnp.dot(x_ref[...], w_ref[...],
                         preferred_element_type=jnp.float32) * s_ref[...]


def _mm_body(x_ref, w_ref, o_ref):
    o_ref[...] = jnp.dot(x_ref[...], w_ref[...],
                         preferred_element_type=jnp.float32)


def _gcn_post_body(p0_ref, p1_ref, s_ref, b_ref, g_ref, bb_ref, res_ref, o_ref):
    t = (p0_ref[0] + p1_ref[0]) * s_ref[...] + b_ref[...]
    o_ref[...] = jnp.maximum(t * g_ref[...] + bb_ref[...], 0.0) + res_ref[...]


def _gat_post_body(pa0_ref, pa1_ref, pb0_ref, pb1_ref, b_ref, g_ref, bb_ref,
                   res_ref, o_ref):
    t = (pa0_ref[0] + pa1_ref[0] + pb0_ref[0] + pb1_ref[0]) * 0.5 + b_ref[...]
    o_ref[...] = jnp.maximum(t * g_ref[...] + bb_ref[...], 0.0) + res_ref[...]


def _rb_spec():
    return pl.BlockSpec((RB, H), lambda i: (i, 0))


def _part_spec(which):
    return pl.BlockSpec((1, RB, H), lambda i, w=which: (w, i, 0))


def _vec_spec():
    return pl.BlockSpec((1, H), lambda i: (0, 0))


def _tc_mm_scale(x, W, s):
    return pl.pallas_call(
        _mm_scale_body,
        grid=(GRID,),
        in_specs=[_rb_spec(),
                  pl.BlockSpec((H, H), lambda i: (0, 0)),
                  _rb_spec()],
        out_specs=_rb_spec(),
        out_shape=jax.ShapeDtypeStruct((NPAD, H), jnp.float32),
    )(x, W, s)


def _tc_mm2(x, W):
    return pl.pallas_call(
        _mm_body,
        grid=(GRID,),
        in_specs=[_rb_spec(),
                  pl.BlockSpec((H, 2 * H), lambda i: (0, 0))],
        out_specs=pl.BlockSpec((RB, 2 * H), lambda i: (i, 0)),
        out_shape=jax.ShapeDtypeStruct((NPAD, 2 * H), jnp.float32),
    )(x, W)


def _tc_gcn_post(parts, s, b, g, bb, res):
    return pl.pallas_call(
        _gcn_post_body,
        grid=(GRID,),
        in_specs=[_part_spec(0), _part_spec(1), _rb_spec(),
                  _vec_spec(), _vec_spec(), _vec_spec(), _rb_spec()],
        out_specs=_rb_spec(),
        out_shape=jax.ShapeDtypeStruct((NPAD, H), jnp.float32),
    )(parts, parts, s, b.reshape(1, H), g.reshape(1, H), bb.reshape(1, H), res)


def _tc_gat_post(pa, pb, b, g, bb, res):
    return pl.pallas_call(
        _gat_post_body,
        grid=(GRID,),
        in_specs=[_part_spec(0), _part_spec(1), _part_spec(0), _part_spec(1),
                  _vec_spec(), _vec_spec(), _vec_spec(), _rb_spec()],
        out_specs=_rb_spec(),
        out_shape=jax.ShapeDtypeStruct((NPAD, H), jnp.float32),
    )(pa, pa, pb, pb, b.reshape(1, H), g.reshape(1, H), bb.reshape(1, H), res)


def _mlp_body(xc_ref, cW1_ref, cb1_ref, cbn_g_ref, cbn_b_ref, cW2_ref,
              cb2_ref, cln_g_ref, cln_b_ref, cW3_ref, cb3_ref, out_ref):
    xc = xc_ref[...]
    z = jnp.dot(xc, cW1_ref[...], preferred_element_type=jnp.float32) + cb1_ref[...]
    z = z * (cbn_g_ref[...] / jnp.sqrt(1.0 + 1e-5)) + cbn_b_ref[...]
    z = jnp.maximum(z, 0.0)
    z = jnp.dot(z, cW2_ref[...], preferred_element_type=jnp.float32) + cb2_ref[...]
    m = jnp.mean(z, axis=-1, keepdims=True)
    v = jnp.mean(jnp.square(z - m), axis=-1, keepdims=True)
    z = (z - m) / jnp.sqrt(v + 1e-5) * cln_g_ref[...] + cln_b_ref[...]
    z = jnp.maximum(z, 0.0)
    out_ref[...] = jnp.dot(z, cW3_ref[...], preferred_element_type=jnp.float32) + cb3_ref[...]


def _alpha(als, ald, row2, col2):
    e = jax.nn.leaky_relu(als[row2] + ald[col2], 0.2)
    emax = jax.ops.segment_max(e, col2, num_segments=N)
    ex = jnp.exp(e - emax[col2])
    den = jax.ops.segment_sum(ex, col2, num_segments=N)
    return ex / (den[col2] + 1e-16)


def _wpack(a):
    a = jnp.pad(a, (0, EPAD - E2))
    return jnp.broadcast_to(a[:, None], (EPAD, 16)).reshape(NW, NCH, K, 16)


def _gat_layer(h, Wg, asrc, adst, bg, g, bb, row2, col2, rowm, colm):
    h2 = _tc_mm2(h, Wg)
    h0 = h2[:, :H]
    h1 = h2[:, H:]
    als0 = jnp.sum(h0[:N] * asrc[0], axis=1)
    ald0 = jnp.sum(h0[:N] * adst[0], axis=1)
    als1 = jnp.sum(h1[:N] * asrc[1], axis=1)
    ald1 = jnp.sum(h1[:N] * adst[1], axis=1)
    a0 = _alpha(als0, ald0, row2, col2)
    a1 = _alpha(als1, ald1, row2, col2)
    pa = _sc_agg_w(h0, rowm, colm, _wpack(a0))
    pb = _sc_agg_w(h1, rowm, colm, _wpack(a1))
    return _tc_gat_post(pa, pb, bg, g, bb, h)


def kernel(x, edge_index, batch, W0, b0, Wg1, asrc1, adst1, bg1, W2c, b2c,
           Wg3, asrc3, adst3, bg3, W4, b4, bn_gamma, bn_beta, cW1, cb1,
           cbn_g, cbn_b, cW2, cb2, cln_g, cln_b, cW3, cb3):
    loop = jnp.arange(N, dtype=edge_index.dtype)
    row2 = jnp.concatenate([edge_index[0], loop])
    col2 = jnp.concatenate([edge_index[1], loop])
    rowm = jnp.pad(row2, (0, EPAD - E2), constant_values=N).reshape(NW, NCH, K)
    colm = jnp.pad(col2, (0, EPAD - E2), constant_values=N).reshape(NW, NCH, 1, K)

    deg = jnp.zeros((N,), jnp.float32).at[col2].add(1.0)
    dis = jnp.where(deg > 0, lax.rsqrt(deg), 0.0)
    dis_b = jnp.broadcast_to(jnp.pad(dis, (0, NPAD - N))[:, None], (NPAD, H))

    xp = jnp.pad(x, ((0, NPAD - N), (0, 0)))
    gn = bn_gamma / jnp.sqrt(1.0 + 1e-5)
    zeros_res = jnp.zeros((NPAD, H), jnp.float32)

    hp = _tc_mm_scale(xp, W0, dis_b)
    parts = _sc_agg(hp, rowm, colm)
    h = _tc_gcn_post(parts, dis_b, b0, gn[0], bn_beta[0], zeros_res)
    h = _gat_layer(h, Wg1, asrc1, adst1, bg1, gn[1], bn_beta[1],
                   row2, col2, rowm, colm)
    hp = _tc_mm_scale(h, W2c, dis_b)
    parts = _sc_agg(hp, rowm, colm)
    h = _tc_gcn_post(parts, dis_b, b2c, gn[2], bn_beta[2], h)
    h = _gat_layer(h, Wg3, asrc3, adst3, bg3, gn[3], bn_beta[3],
                   row2, col2, rowm, colm)
    hp = _tc_mm_scale(h, W4, dis_b)
    parts = _sc_agg(hp, rowm, colm)
    h = _tc_gcn_post(parts, dis_b, b4, gn[4], bn_beta[4], h)

    hN = h[:N]
    counts = jax.ops.segment_sum(jnp.ones((N,), jnp.float32), batch,
                                 num_segments=G, indices_are_sorted=True)
    xmean = jax.ops.segment_sum(hN, batch, num_segments=G,
                                indices_are_sorted=True)
    xmean = xmean / jnp.maximum(counts, 1.0)[:, None]
    xmax = jax.ops.segment_max(hN, batch, num_segments=G,
                               indices_are_sorted=True)
    xc = jnp.concatenate([xmean, xmax], axis=1)

    return pl.pallas_call(
        _mlp_body,
        out_shape=jax.ShapeDtypeStruct((G, 2), jnp.float32),
    )(xc, cW1, cb1, cbn_g, cbn_b, cW2, cb2, cln_g, cln_b, cW3, cb3)


# speedup vs baseline: 23.5475x; 9.2693x over previous
import functools

import jax
import jax.numpy as jnp
import numpy as _np
from jax import lax
from jax.experimental import pallas as pl
from jax.experimental.pallas import tpu as pltpu
from jax.experimental.pallas import tpu_sc as plsc

N = 10000
E = 320000
H = 128
HEADS = 2
G = 64

NPAD = 10240
E2 = E + N
NC, NS = 2, 16
NW = NC * NS
K = 128
NCH = -(-E2 // (NW * K))
EPT = NCH * K
EPAD = NW * EPT
STRIPE = NPAD // NS
RB = 512
GRID = NPAD // RB

_i32 = jnp.int32

_mesh = plsc.VectorSubcoreMesh(core_axis_name="c", subcore_axis_name="s")


def _zero_acc(buf, acc, sid):
    zero16 = jnp.zeros((16,), jnp.float32)

    def zbody(r, carry):
        for c in range(H // 16):
            buf[r, pl.ds(c * 16, 16)] = zero16
        return carry

    lax.fori_loop(0, K, zbody, 0)
    for kk in range(STRIPE // K):
        pltpu.sync_copy(buf, acc.at[pl.ds(sid * STRIPE + kk * K, K)])


@functools.partial(
    pl.kernel,
    out_type=jax.ShapeDtypeStruct((NC, NPAD, H), jnp.float32),
    mesh=_mesh,
    scratch_types=[
        pltpu.VMEM((K,), _i32),
        pltpu.VMEM((1, K), _i32),
        pltpu.VMEM((K, H), jnp.float32),
        pltpu.VMEM_SHARED((NPAD, H), jnp.float32),
        pltpu.SemaphoreType.DMA,
    ],
)
def _sc_agg(src_hbm, row_hbm, col_hbm, out_hbm, row_c, col_c, buf, acc, sem):
    cid = lax.axis_index("c")
    sid = lax.axis_index("s")
    wid = sid * NC + cid
    _zero_acc(buf, acc, sid)
    plsc.subcore_barrier()

    def body(j, carry):
        pltpu.sync_copy(row_hbm.at[wid, j], row_c)
        pltpu.sync_copy(col_hbm.at[wid, j], col_c)
        pltpu.async_copy(src_hbm.at[row_c], buf, sem).wait()
        pltpu.sync_copy(buf, acc.at[col_c.at[0]], add=True)
        return carry

    lax.fori_loop(0, NCH, body, 0)
    plsc.subcore_barrier()
    pltpu.sync_copy(acc.at[pl.ds(sid * STRIPE, STRIPE)],
                    out_hbm.at[cid, pl.ds(sid * STRIPE, STRIPE)])


@functools.partial(
    pl.kernel,
    out_type=jax.ShapeDtypeStruct((NC, NPAD, H), jnp.float32),
    mesh=_mesh,
    scratch_types=[
        pltpu.VMEM((K,), _i32),
        pltpu.VMEM((1, K), _i32),
        pltpu.VMEM((K, 16), jnp.float32),
        pltpu.VMEM((K, H), jnp.float32),
        pltpu.VMEM_SHARED((NPAD, H), jnp.float32),
        pltpu.SemaphoreType.DMA,
    ],
)
def _sc_agg_w(src_hbm, row_hbm, col_hbm, w_hbm, out_hbm,
              row_c, col_c, wbuf, buf, acc, sem):
    cid = lax.axis_index("c")
    sid = lax.axis_index("s")
    wid = sid * NC + cid
    _zero_acc(buf, acc, sid)
    plsc.subcore_barrier()

    def body(j, carry):
        pltpu.sync_copy(row_hbm.at[wid, j], row_c)
        pltpu.sync_copy(col_hbm.at[wid, j], col_c)
        pltpu.sync_copy(w_hbm.at[wid, j], wbuf)
        pltpu.async_copy(src_hbm.at[row_c], buf, sem).wait()

        def rowbody(r, c2):
            wspl = wbuf[r]
            for c in range(H // 16):
                sl = pl.ds(c * 16, 16)
                buf[r, sl] = buf[r, sl] * wspl
            return c2

        lax.fori_loop(0, K, rowbody, 0)
        pltpu.sync_copy(buf, acc.at[col_c.at[0]], add=True)
        return carry

    lax.fori_loop(0, NCH, body, 0)
    plsc.subcore_barrier()
    pltpu.sync_copy(acc.at[pl.ds(sid * STRIPE, STRIPE)],
                    out_hbm.at[cid, pl.ds(sid * STRIPE, STRIPE)])


def _zero1d(zb, acc, sid):
    zero16 = jnp.zeros((16,), jnp.float32)
    for c in range(K // 16):
        zb[pl.ds(c * 16, 16)] = zero16
    for kk in range(STRIPE // K):
        pltpu.sync_copy(zb, acc.at[pl.ds(sid * STRIPE + kk * K, K)])


@functools.partial(
    pl.kernel,
    out_type=(jax.ShapeDtypeStruct((NW, NCH, K), jnp.float32),
              jax.ShapeDtypeStruct((NW, NCH, K), jnp.float32),
              jax.ShapeDtypeStruct((NC, NPAD), jnp.float32),
              jax.ShapeDtypeStruct((NC, NPAD), jnp.float32)),
    mesh=_mesh,
    scratch_types=[
        pltpu.VMEM((K,), _i32),
        pltpu.VMEM((1, K), _i32),
        pltpu.VMEM((K,), jnp.float32),
        pltpu.VMEM((K,), jnp.float32),
        pltpu.VMEM((K,), jnp.float32),
        pltpu.VMEM((K,), jnp.float32),
        pltpu.VMEM((K,), jnp.float32),
        pltpu.VMEM((K,), jnp.float32),
        pltpu.VMEM((K,), jnp.float32),
        pltpu.VMEM((K,), jnp.float32),
        pltpu.VMEM_SHARED((NPAD,), jnp.float32),
        pltpu.VMEM_SHARED((NPAD,), jnp.float32),
        pltpu.SemaphoreType.DMA,
    ],
)
def _sc_edge_ex(als0_h, ald0_h, l0_h, als1_h, ald1_h, l1_h, row_hbm, col_hbm,
                ex0_hbm, ex1_hbm, den0_hbm, den1_hbm,
                row_c, col_c, a0b, b0b, c0b, a1b, b1b, c1b, e0b, e1b,
                dacc0, dacc1, sem):
    cid = lax.axis_index("c")
    sid = lax.axis_index("s")
    wid = sid * NC + cid
    _zero1d(e0b, dacc0, sid)
    _zero1d(e0b, dacc1, sid)
    plsc.subcore_barrier()

    def body(j, carry):
        pltpu.sync_copy(row_hbm.at[wid, j], row_c)
        pltpu.sync_copy(col_hbm.at[wid, j], col_c)
        cps = [
            pltpu.async_copy(als0_h.at[row_c], a0b, sem),
            pltpu.async_copy(ald0_h.at[col_c.at[0]], b0b, sem),
            pltpu.async_copy(l0_h.at[col_c.at[0]], c0b, sem),
            pltpu.async_copy(als1_h.at[row_c], a1b, sem),
            pltpu.async_copy(ald1_h.at[col_c.at[0]], b1b, sem),
            pltpu.async_copy(l1_h.at[col_c.at[0]], c1b, sem),
        ]
        for cp in cps:
            cp.wait()
        for rb in range(K // 16):
            sl = pl.ds(rb * 16, 16)
            s = a0b[sl] + b0b[sl]
            s = jnp.where(s > 0, s, s * 0.2)
            e0b[sl] = jnp.exp(jnp.minimum(s - c0b[sl], 80.0))
            t = a1b[sl] + b1b[sl]
            t = jnp.where(t > 0, t, t * 0.2)
            e1b[sl] = jnp.exp(jnp.minimum(t - c1b[sl], 80.0))
        pltpu.sync_copy(e0b, ex0_hbm.at[wid, j])
        pltpu.sync_copy(e1b, ex1_hbm.at[wid, j])
        pltpu.sync_copy(e0b, dacc0.at[col_c.at[0]], add=True)
        pltpu.sync_copy(e1b, dacc1.at[col_c.at[0]], add=True)
        return carry

    lax.fori_loop(0, NCH, body, 0)
    plsc.subcore_barrier()
    pltpu.sync_copy(dacc0.at[pl.ds(sid * STRIPE, STRIPE)],
                    den0_hbm.at[cid, pl.ds(sid * STRIPE, STRIPE)])
    pltpu.sync_copy(dacc1.at[pl.ds(sid * STRIPE, STRIPE)],
                    den1_hbm.at[cid, pl.ds(sid * STRIPE, STRIPE)])


@functools.partial(
    pl.kernel,
    out_type=jax.ShapeDtypeStruct((NC, NPAD), jnp.float32),
    mesh=_mesh,
    scratch_types=[
        pltpu.VMEM((1, K), _i32),
        pltpu.VMEM((K,), jnp.float32),
        pltpu.VMEM_SHARED((NPAD,), jnp.float32),
    ],
)
def _sc_deg(col_hbm, out_hbm, col_c, oneb, dacc):
    cid = lax.axis_index("c")
    sid = lax.axis_index("s")
    wid = sid * NC + cid
    _zero1d(oneb, dacc, sid)
    one16 = jnp.full((16,), 1.0, jnp.float32)
    for c in range(K // 16):
        oneb[pl.ds(c * 16, 16)] = one16
    plsc.subcore_barrier()

    def body(j, carry):
        pltpu.sync_copy(col_hbm.at[wid, j], col_c)
        pltpu.sync_copy(oneb, dacc.at[col_c.at[0]], add=True)
        return carry

    lax.fori_loop(0, NCH, body, 0)
    plsc.subcore_barrier()
    pltpu.sync_copy(dacc.at[pl.ds(sid * STRIPE, STRIPE)],
                    out_hbm.at[cid, pl.ds(sid * STRIPE, STRIPE)])


def _mm_scale_body(x_ref, w_ref, s_ref, o_ref):
    o_ref[...] = jnp.dot(x_ref[...], w_ref[...],
                         preferred_element_type=jnp.float32) * s_ref[...]


def _mm_body(x_ref, w_ref, o_ref):
    o_ref[...] = jnp.dot(x_ref[...], w_ref[...],
                         preferred_element_type=jnp.float32)


def _gcn_post_body(p0_ref, p1_ref, s_ref, b_ref, g_ref, bb_ref, res_ref, o_ref):
    t = (p0_ref[0] + p1_ref[0]) * s_ref[...] + b_ref[...]
    o_ref[...] = jnp.maximum(t * g_ref[...] + bb_ref[...], 0.0) + res_ref[...]


def _gat_post_body(pa0_ref, pa1_ref, pb0_ref, pb1_ref, ra_ref, rb_ref,
                   b_ref, g_ref, bb_ref, res_ref, o_ref):
    t = ((pa0_ref[0] + pa1_ref[0]) * ra_ref[...]
         + (pb0_ref[0] + pb1_ref[0]) * rb_ref[...]) * 0.5 + b_ref[...]
    o_ref[...] = jnp.maximum(t * g_ref[...] + bb_ref[...], 0.0) + res_ref[...]


def _rb_spec():
    return pl.BlockSpec((RB, H), lambda i: (i, 0))


def _part_spec(which):
    return pl.BlockSpec((1, RB, H), lambda i, w=which: (w, i, 0))


def _vec_spec():
    return pl.BlockSpec((1, H), lambda i: (0, 0))


def _tc_mm_scale(x, W, s):
    return pl.pallas_call(
        _mm_scale_body,
        grid=(GRID,),
        in_specs=[_rb_spec(),
                  pl.BlockSpec((H, H), lambda i: (0, 0)),
                  _rb_spec()],
        out_specs=_rb_spec(),
        out_shape=jax.ShapeDtypeStruct((NPAD, H), jnp.float32),
    )(x, W, s)


def _tc_mm2(x, W):
    return pl.pallas_call(
        _mm_body,
        grid=(GRID,),
        in_specs=[_rb_spec(),
                  pl.BlockSpec((H, 2 * H), lambda i: (0, 0))],
        out_specs=pl.BlockSpec((RB, 2 * H), lambda i: (i, 0)),
        out_shape=jax.ShapeDtypeStruct((NPAD, 2 * H), jnp.float32),
    )(x, W)


def _tc_gcn_post(parts, s, b, g, bb, res):
    return pl.pallas_call(
        _gcn_post_body,
        grid=(GRID,),
        in_specs=[_part_spec(0), _part_spec(1), _rb_spec(),
                  _vec_spec(), _vec_spec(), _vec_spec(), _rb_spec()],
        out_specs=_rb_spec(),
        out_shape=jax.ShapeDtypeStruct((NPAD, H), jnp.float32),
    )(parts, parts, s, b.reshape(1, H), g.reshape(1, H), bb.reshape(1, H), res)


def _tc_gat_post(pa, pb, ra, rb, b, g, bb, res):
    return pl.pallas_call(
        _gat_post_body,
        grid=(GRID,),
        in_specs=[_part_spec(0), _part_spec(1), _part_spec(0), _part_spec(1),
                  _rb_spec(), _rb_spec(),
                  _vec_spec(), _vec_spec(), _vec_spec(), _rb_spec()],
        out_specs=_rb_spec(),
        out_shape=jax.ShapeDtypeStruct((NPAD, H), jnp.float32),
    )(pa, pa, pb, pb, ra, rb,
      b.reshape(1, H), g.reshape(1, H), bb.reshape(1, H), res)


def _mlp_body(xc_ref, cW1_ref, cb1_ref, cbn_g_ref, cbn_b_ref, cW2_ref,
              cb2_ref, cln_g_ref, cln_b_ref, cW3_ref, cb3_ref, out_ref):
    xc = xc_ref[...]
    z = jnp.dot(xc, cW1_ref[...], preferred_element_type=jnp.float32) + cb1_ref[...]
    z = z * (cbn_g_ref[...] / jnp.sqrt(1.0 + 1e-5)) + cbn_b_ref[...]
    z = jnp.maximum(z, 0.0)
    z = jnp.dot(z, cW2_ref[...], preferred_element_type=jnp.float32) + cb2_ref[...]
    m = jnp.mean(z, axis=-1, keepdims=True)
    v = jnp.mean(jnp.square(z - m), axis=-1, keepdims=True)
    z = (z - m) / jnp.sqrt(v + 1e-5) * cln_g_ref[...] + cln_b_ref[...]
    z = jnp.maximum(z, 0.0)
    out_ref[...] = jnp.dot(z, cW3_ref[...], preferred_element_type=jnp.float32) + cb3_ref[...]


def _wsplat(ex_mat):
    a = ex_mat.reshape(EPAD)
    return jnp.broadcast_to(a[:, None], (EPAD, 16)).reshape(NW, NCH, K, 16)


def _padn(v):
    return jnp.pad(v, (0, NPAD - N))


def _gat_layer(h, Wg, asrc, adst, bg, g, bb, rowm, colm):
    h2 = _tc_mm2(h, Wg)
    h0 = h2[:, :H]
    h1 = h2[:, H:]
    als0 = jnp.sum(h0[:N] * asrc[0], axis=1)
    ald0 = jnp.sum(h0[:N] * adst[0], axis=1)
    als1 = jnp.sum(h1[:N] * asrc[1], axis=1)
    ald1 = jnp.sum(h1[:N] * adst[1], axis=1)
    l0 = jax.nn.leaky_relu(als0 + ald0, 0.2)
    l1 = jax.nn.leaky_relu(als1 + ald1, 0.2)
    ex0, ex1, dp0, dp1 = _sc_edge_ex(
        _padn(als0), _padn(ald0), _padn(l0),
        _padn(als1), _padn(ald1), _padn(l1), rowm, colm)
    rd0 = 1.0 / (dp0[0] + dp0[1] + 1e-16)
    rd1 = 1.0 / (dp1[0] + dp1[1] + 1e-16)
    ra = jnp.broadcast_to(rd0[:, None], (NPAD, H))
    rb = jnp.broadcast_to(rd1[:, None], (NPAD, H))
    pa = _sc_agg_w(h0, rowm, colm, _wsplat(ex0))
    pb = _sc_agg_w(h1, rowm, colm, _wsplat(ex1))
    return _tc_gat_post(pa, pb, ra, rb, bg, g, bb, h)


def kernel(x, edge_index, batch, W0, b0, Wg1, asrc1, adst1, bg1, W2c, b2c,
           Wg3, asrc3, adst3, bg3, W4, b4, bn_gamma, bn_beta, cW1, cb1,
           cbn_g, cbn_b, cW2, cb2, cln_g, cln_b, cW3, cb3):
    loop = jnp.arange(N, dtype=edge_index.dtype)
    row2 = jnp.concatenate([edge_index[0], loop])
    col2 = jnp.concatenate([edge_index[1], loop])
    rowm = jnp.pad(row2, (0, EPAD - E2), constant_values=N).reshape(NW, NCH, K)
    colm = jnp.pad(col2, (0, EPAD - E2), constant_values=N).reshape(NW, NCH, 1, K)

    degp = _sc_deg(colm)
    deg = (degp[0] + degp[1])[:N]
    dis = jnp.where(deg > 0, lax.rsqrt(deg), 0.0)
    dis_b = jnp.broadcast_to(jnp.pad(dis, (0, NPAD - N))[:, None], (NPAD, H))

    xp = jnp.pad(x, ((0, NPAD - N), (0, 0)))
    gn = bn_gamma / jnp.sqrt(1.0 + 1e-5)
    zeros_res = jnp.zeros((NPAD, H), jnp.float32)

    hp = _tc_mm_scale(xp, W0, dis_b)
    parts = _sc_agg(hp, rowm, colm)
    h = _tc_gcn_post(parts, dis_b, b0, gn[0], bn_beta[0], zeros_res)
    h = _gat_layer(h, Wg1, asrc1, adst1, bg1, gn[1], bn_beta[1], rowm, colm)
    hp = _tc_mm_scale(h, W2c, dis_b)
    parts = _sc_agg(hp, rowm, colm)
    h = _tc_gcn_post(parts, dis_b, b2c, gn[2], bn_beta[2], h)
    h = _gat_layer(h, Wg3, asrc3, adst3, bg3, gn[3], bn_beta[3], rowm, colm)
    hp = _tc_mm_scale(h, W4, dis_b)
    parts = _sc_agg(hp, rowm, colm)
    h = _tc_gcn_post(parts, dis_b, b4, gn[4], bn_beta[4], h)

    hN = h[:N]
    counts = jax.ops.segment_sum(jnp.ones((N,), jnp.float32), batch,
                                 num_segments=G, indices_are_sorted=True)
    xmean = jax.ops.segment_sum(hN, batch, num_segments=G,
                                indices_are_sorted=True)
    xmean = xmean / jnp.maximum(counts, 1.0)[:, None]
    xmax = jax.ops.segment_max(hN, batch, num_segments=G,
                               indices_are_sorted=True)
    xc = jnp.concatenate([xmean, xmax], axis=1)

    return pl.pallas_call(
        _mlp_body,
        out_shape=jax.ShapeDtypeStruct((G, 2), jnp.float32),
    )(xc, cW1, cb1, cbn_g, cbn_b, cW2, cb2, cln_g, cln_b, cW3, cb3)
